# baseline (speedup 1.0000x reference)
import functools

import jax
import jax.numpy as jnp
from jax import lax
from jax.experimental import pallas as pl
from jax.experimental.pallas import tpu as pltpu
from jax.experimental.pallas import tpu_sc as plsc

_LANES = 16


@functools.lru_cache(maxsize=None)
def _build(B, T, V, D):
    info = plsc.get_sparse_core_info()
    NC, NS = info.num_cores, info.num_subcores
    NW = NC * NS
    assert B % NW == 0
    rows_per_w = B // NW

    g_chunk = 40
    assert T % g_chunk == 0
    n_g = T // g_chunk

    mesh = plsc.VectorSubcoreMesh(core_axis_name="c", subcore_axis_name="s")

    @functools.partial(
        pl.kernel,
        mesh=mesh,
        out_type=jax.ShapeDtypeStruct((B * T, D), jnp.float32),
        compiler_params=pltpu.CompilerParams(use_tc_tiling_on_sc=False),
        scratch_types=[
            pltpu.VMEM((T,), jnp.int32),
            pltpu.VMEM((T, D), jnp.float32),
            pltpu.VMEM((T, D), jnp.float32),
            pltpu.SemaphoreType.DMA,
        ],
    )
    def emb(tok_hbm, tab_hbm, pos_hbm, out_hbm, idx_v, pos_v, buf_v, sem):
        wid = lax.axis_index("s") * NC + lax.axis_index("c")
        pltpu.sync_copy(pos_hbm, pos_v)

        def row_body(i, carry):
            r = wid * rows_per_w + i
            pltpu.sync_copy(tok_hbm.at[pl.ds(r * T, T)], idx_v)
            for j in range(n_g):
                sl = pl.ds(j * g_chunk, g_chunk)
                pltpu.async_copy(tab_hbm.at[idx_v.at[sl]], buf_v.at[sl], sem).wait()

            def add_body(rr, c2):
                for c in range(D // _LANES):
                    sl = pl.ds(c * _LANES, _LANES)
                    buf_v[rr, sl] = buf_v[rr, sl] + pos_v[rr, sl]
                return c2

            lax.fori_loop(0, T, add_body, 0, unroll=False)
            pltpu.sync_copy(buf_v, out_hbm.at[pl.ds(r * T, T)])
            return carry

        lax.fori_loop(0, rows_per_w, row_body, 0, unroll=False)

    return emb


def kernel(tokens, token_embeddings, position_embeddings):
    B, T = tokens.shape
    V, D = token_embeddings.shape
    emb = _build(B, T, V, D)
    tok_flat = tokens.reshape(B * T).astype(jnp.int32)
    out = emb(tok_flat, token_embeddings, position_embeddings)
    return out.reshape(B, T, D)


# speedup vs baseline: 1.4427x; 1.4427x over previous
import functools

import jax
import jax.numpy as jnp
from jax import lax
from jax.experimental import pallas as pl
from jax.experimental.pallas import tpu as pltpu
from jax.experimental.pallas import tpu_sc as plsc

_LANES = 16
_NBUF = 4


@functools.lru_cache(maxsize=None)
def _build(B, T, V, D):
    info = plsc.get_sparse_core_info()
    NC, NS = info.num_cores, info.num_subcores
    NW = NC * NS
    assert B % NW == 0
    rows_per_w = B // NW
    idx_per_w = rows_per_w * T

    chunks = []
    off = 0
    while off < T:
        n = min(128, T - off)
        chunks.append((off, n))
        off += n
    row_bytes = T * D * 4

    mesh = plsc.VectorSubcoreMesh(core_axis_name="c", subcore_axis_name="s")

    @functools.partial(
        pl.kernel,
        mesh=mesh,
        out_type=jax.ShapeDtypeStruct((B * T, D), jnp.float32),
        compiler_params=pltpu.CompilerParams(use_tc_tiling_on_sc=False),
        scratch_types=[
            pltpu.VMEM((idx_per_w,), jnp.int32),
            pltpu.VMEM((T, D), jnp.float32),
            pltpu.VMEM((_NBUF, T, D), jnp.float32),
        ]
        + [pltpu.SemaphoreType.DMA] * (2 * _NBUF),
    )
    def emb(tok_hbm, tab_hbm, pos_hbm, out_hbm, idx_v, pos_v, buf_v, *sems):
        sem_g = sems[:_NBUF]
        sem_s = sems[_NBUF:]
        wid = lax.axis_index("s") * NC + lax.axis_index("c")
        pltpu.sync_copy(pos_hbm, pos_v)
        pltpu.sync_copy(tok_hbm.at[pl.ds(wid * idx_per_w, idx_per_w)], idx_v)
        dummy = out_hbm.at[pl.ds(0, T)]

        def fire_gather(r, bf):
            for off, n in chunks:
                pltpu.async_copy(
                    tab_hbm.at[idx_v.at[pl.ds(r * T + off, n)]],
                    buf_v.at[bf, pl.ds(off, n)],
                    sem_g[bf],
                )

        def drain_gather(b):
            pltpu.make_async_copy(dummy, buf_v.at[b], sem_g[b]).wait()

        def add_pos(b):
            def body(rr, c):
                for cc in range(D // _LANES):
                    sl = pl.ds(cc * _LANES, _LANES)
                    plsc.addupdate(buf_v.at[b, rr, sl], pos_v[rr, sl])
                return c

            lax.fori_loop(0, T, body, 0, unroll=2)

        def fire_store(r, b):
            gr = wid * rows_per_w + r
            pltpu.async_copy(buf_v.at[b], out_hbm.at[pl.ds(gr * T, T)], sem_s[b])

        def drain_store(b):
            pltpu.make_async_copy(buf_v.at[b], dummy, sem_s[b]).wait()

        fire_gather(0, 0)
        fire_gather(1, 1)
        for r in (0, 1):
            fire_gather(r + 2, r + 2)
            drain_gather(r)
            add_pos(r)
            fire_store(r, r)

        def body(g, c):
            for k in range(_NBUF):
                r = _NBUF * g + 2 + k
                b = (2 + k) % _NBUF
                bf = k
                drain_store(bf)
                fire_gather(r + 2, bf)
                drain_gather(b)
                add_pos(b)
                fire_store(r, b)
            return c

        n_steady = (rows_per_w - 4) // _NBUF
        lax.fori_loop(0, n_steady, body, 0, unroll=False)

        for r in (rows_per_w - 2, rows_per_w - 1):
            b = r % _NBUF
            drain_gather(b)
            add_pos(b)
            fire_store(r, b)
        for b in range(_NBUF):
            drain_store(b)

    return emb


def kernel(tokens, token_embeddings, position_embeddings):
    B, T = tokens.shape
    V, D = token_embeddings.shape
    emb = _build(B, T, V, D)
    tok_flat = tokens.reshape(B * T).astype(jnp.int32)
    out = emb(tok_flat, token_embeddings, position_embeddings)
    return out.reshape(B, T, D)
